# Initial kernel scaffold; baseline (speedup 1.0000x reference)
import jax
import jax.numpy as jnp
from jax.experimental import pallas as pl


def kernel(feat0, feat1, feat2, feat3, feat4, rpn_conv_w, rpn_conv_b, rpn_cls_w, rpn_cls_b, rpn_box_w, rpn_box_b, fc1_w, fc1_b, fc2_w, fc2_b, cls_w, cls_b, reg_w, reg_b):
    raise NotImplementedError("write your pallas kernel here")


# speedup vs baseline: 3.1972x; 3.1972x over previous
import functools

import jax
import jax.numpy as jnp
import numpy as np
from jax import lax
from jax.experimental import pallas as pl
from jax.experimental.pallas import tpu as pltpu
from jax.experimental.pallas import tpu_sc as plsc

STRIDES = [4, 8, 16, 32, 64]
RATIOS = [0.5, 1.0, 2.0]
SCALE = 8
A = 3
IMG = 512
C = 256
NUM_CLASSES = 80
PRE_NMS = 1000
POST_NMS = 1000
IOU_THR = 0.7
DEF = jax.lax.Precision.DEFAULT

_LEVEL_HW = [(IMG // s, IMG // s) for s in STRIDES]
_LEVEL_N = [h * w * A for h, w in _LEVEL_HW]
_LEVEL_K = [min(PRE_NMS, n) for n in _LEVEL_N]
_NCAND = sum(_LEVEL_K)
_NMS_ROWS = 31


def _dot(a, b):
    return lax.dot_general(a, b, (((1,), (0,)), ((), ())), precision=DEF,
                           preferred_element_type=jnp.float32)


def _make_anchor_params(H, W, stride):
    size = np.float32(SCALE * stride)
    r = np.array(RATIOS, dtype=np.float32)
    hr = np.sqrt(r)
    wr = (np.float32(1.0) / hr).astype(np.float32)
    ws = (size * wr).astype(np.float32)
    hs = (size * hr).astype(np.float32)
    xs = ((np.arange(W, dtype=np.float32) + np.float32(0.5)) * np.float32(stride)).astype(np.float32)
    ys = ((np.arange(H, dtype=np.float32) + np.float32(0.5)) * np.float32(stride)).astype(np.float32)
    cx, cy = np.meshgrid(xs, ys)
    cx = cx[:, :, None].astype(np.float32)
    cy = cy[:, :, None].astype(np.float32)
    x1 = (cx - ws[None, None, :] / np.float32(2)).astype(np.float32)
    y1 = (cy - hs[None, None, :] / np.float32(2)).astype(np.float32)
    x2 = (cx + ws[None, None, :] / np.float32(2)).astype(np.float32)
    y2 = (cy + hs[None, None, :] / np.float32(2)).astype(np.float32)
    pw = (x2 - x1).astype(np.float32)
    ph = (y2 - y1).astype(np.float32)
    px = ((x1 + x2) * np.float32(0.5)).astype(np.float32)
    py = ((y1 + y2) * np.float32(0.5)).astype(np.float32)
    out = np.zeros((H * W, 16), np.float32)
    for a in range(A):
        out[:, a * 4 + 0] = pw[:, :, a].reshape(-1)
        out[:, a * 4 + 1] = ph[:, :, a].reshape(-1)
        out[:, a * 4 + 2] = px[:, :, a].reshape(-1)
        out[:, a * 4 + 3] = py[:, :, a].reshape(-1)
    return jnp.asarray(out)


def _rpn_level(x2, w9, b, wcb, bcb, anc, M, bm):
    def body(x_ref, w_ref, b_ref, wcb_ref, bcb_ref, anc_ref, o_ref):
        h = jnp.maximum(_dot(x_ref[...], w_ref[...]) + b_ref[...], 0.0)
        scb = _dot(h, wcb_ref[...]) + bcb_ref[...]
        groups = []
        zero = jnp.zeros((scb.shape[0], 123), jnp.float32)
        for a in range(A):
            g = 16 * a
            logit = scb[:, g:g + 1]
            score = 1.0 / (1.0 + jnp.exp(-logit))
            dxv = scb[:, g + 1:g + 2]
            dyv = scb[:, g + 2:g + 3]
            dwv = scb[:, g + 3:g + 4]
            dhv = scb[:, g + 4:g + 5]
            pw = anc_ref[:, 4 * a + 0:4 * a + 1]
            ph = anc_ref[:, 4 * a + 1:4 * a + 2]
            px = anc_ref[:, 4 * a + 2:4 * a + 3]
            py = anc_ref[:, 4 * a + 3:4 * a + 4]
            gx = px + pw * dxv
            gy = py + ph * dyv
            gw = pw * jnp.exp(jnp.clip(dwv, -4.0, 4.0))
            gh = ph * jnp.exp(jnp.clip(dhv, -4.0, 4.0))
            bx1 = jnp.clip(gx - gw * 0.5, 0.0, float(IMG))
            by1 = jnp.clip(gy - gh * 0.5, 0.0, float(IMG))
            bx2 = jnp.clip(gx + gw * 0.5, 0.0, float(IMG))
            by2 = jnp.clip(gy + gh * 0.5, 0.0, float(IMG))
            groups.append(jnp.concatenate(
                [score, bx1, by1, bx2, by2, zero], axis=1))
        o_ref[...] = jnp.concatenate(groups, axis=1)

    return pl.pallas_call(
        body,
        grid=(M // bm,),
        in_specs=[
            pl.BlockSpec((bm, 9 * C), lambda i: (i, 0)),
            pl.BlockSpec((9 * C, C), lambda i: (0, 0)),
            pl.BlockSpec((1, C), lambda i: (0, 0)),
            pl.BlockSpec((C, 48), lambda i: (0, 0)),
            pl.BlockSpec((1, 48), lambda i: (0, 0)),
            pl.BlockSpec((bm, 16), lambda i: (i, 0)),
        ],
        out_specs=pl.BlockSpec((bm, 384), lambda i: (i, 0)),
        out_shape=jax.ShapeDtypeStruct((M, 384), jnp.float32),
    )(x2, w9, b, wcb, bcb, anc)


def _topk_dest(scores2d, ut, su, k):
    Nr = scores2d.shape[0]

    def body(s_ref, ut_ref, su_ref, o_ref):
        bits = lax.bitcast_convert_type(s_ref[...], jnp.int32)

        def cnt_gt(t):
            return jnp.sum((bits > t).astype(jnp.int32))

        def step(_, lr):
            lo, hi = lr
            mid = (lo + hi) // 2
            pred = cnt_gt(mid) < k
            return (jnp.where(pred, lo, mid + 1), jnp.where(pred, mid, hi))

        lo, hi = lax.fori_loop(0, 31, step, (jnp.int32(0), jnp.int32(1 << 30)))
        t = hi
        mgt = bits > t
        meq = bits == t
        m = jnp.sum(mgt.astype(jnp.int32))
        fgt = mgt.astype(jnp.float32)
        feq = meq.astype(jnp.float32)
        pg_in = _dot(fgt, ut_ref[...])
        pe_in = _dot(feq, ut_ref[...])
        rg = pg_in[:, 127:128]
        re = pe_in[:, 127:128]
        pg_row = _dot(su_ref[...], rg)
        pe_row = _dot(su_ref[...], re)
        dest_gt = (pg_in + pg_row - 1.0).astype(jnp.int32)
        dest_eq = (pe_in + pe_row - 1.0).astype(jnp.int32) + m
        dest = jnp.where(mgt, dest_gt,
                         jnp.where(meq & (dest_eq < k), dest_eq, k))
        o_ref[...] = dest

    return pl.pallas_call(
        body,
        out_shape=jax.ShapeDtypeStruct((Nr, 128), jnp.int32),
    )(scores2d, ut, su)


def _sc_compact(rows, dest, k):
    N = rows.shape[0]
    NW = 32
    chunk = N // NW
    sub = min(chunk, 128)
    nsub = chunk // sub
    mesh = plsc.VectorSubcoreMesh(core_axis_name="c", subcore_axis_name="s")
    dest2d = dest.reshape(NW, nsub, sub)

    @functools.partial(
        pl.kernel, mesh=mesh,
        out_type=jax.ShapeDtypeStruct((k + 8, 128), jnp.float32),
        scratch_types=[
            pltpu.VMEM((nsub, sub), jnp.int32),
            pltpu.VMEM((sub, 128), jnp.float32),
            pltpu.SemaphoreType.DMA,
        ],
    )
    def kk(rows_hbm, dest_hbm, out_hbm, idx_v, rows_v, sem):
        wid = lax.axis_index("s") * 2 + lax.axis_index("c")
        base = wid * chunk
        pltpu.sync_copy(dest_hbm.at[wid], idx_v)
        for j in range(nsub):
            pltpu.sync_copy(rows_hbm.at[pl.ds(base + j * sub, sub), :], rows_v)
            pltpu.async_copy(rows_v, out_hbm.at[idx_v.at[j]], sem).wait()

    return kk(rows, dest2d)


def _nms_pallas(cand16, s2d, x12d, y12d, x22d, y22d):
    def body(cand_ref, s_ref, x1_ref, y1_ref, x2_ref, y2_ref,
             props_ref, ksc_ref, srun_ref):
        x1 = x1_ref[...]
        y1 = y1_ref[...]
        x2 = x2_ref[...]
        y2 = y2_ref[...]
        areas = (x2 - x1) * (y2 - y1)
        srun_ref[...] = s_ref[...]
        flat = (lax.broadcasted_iota(jnp.int32, (_NMS_ROWS, 128), 0) * 128
                + lax.broadcasted_iota(jnp.int32, (_NMS_ROWS, 128), 1))
        big = jnp.int32(1 << 28)
        neginf = jnp.float32(-jnp.inf)
        s0 = jnp.where(flat < _LEVEL_K[0], s_ref[...], neginf)
        m0 = jnp.max(s0)
        p0 = jnp.min(jnp.where(s0 == m0, flat, big))

        def step(i, _):
            s = srun_ref[...]
            m = jnp.max(s)
            idx = jnp.min(jnp.where(s == m, flat, big))
            idx = jnp.where(m == neginf, p0, idx)
            row = cand_ref[pl.ds(idx, 1), :]
            props_ref[pl.ds(i, 1), :] = row[:, 1:5]
            ksc_ref[pl.ds(i, 1), :] = row[:, 0:1]
            bx1 = row[0, 1]
            by1 = row[0, 2]
            bx2 = row[0, 3]
            by2 = row[0, 4]
            barea = (bx2 - bx1) * (by2 - by1)
            xx1 = jnp.maximum(bx1, x1)
            yy1 = jnp.maximum(by1, y1)
            xx2 = jnp.minimum(bx2, x2)
            yy2 = jnp.minimum(by2, y2)
            inter = jnp.maximum(xx2 - xx1, 0.0) * jnp.maximum(yy2 - yy1, 0.0)
            iou = inter / (barea + areas - inter + 1e-6)
            srun_ref[...] = jnp.where(iou >= IOU_THR, neginf, s)
            return 0

        lax.fori_loop(0, POST_NMS, step, 0)

    return pl.pallas_call(
        body,
        out_shape=(jax.ShapeDtypeStruct((POST_NMS, 4), jnp.float32),
                   jax.ShapeDtypeStruct((POST_NMS, 1), jnp.float32)),
        scratch_shapes=[pltpu.VMEM((_NMS_ROWS, 128), jnp.float32)],
    )(cand16, s2d, x12d, y12d, x22d, y22d)


def _roi_pre(props, rmat, tmat):
    W = _LEVEL_HW[0][1]

    def body(p_ref, r_ref, t_ref, i00, i01, i10, i11, o00, o01, o10, o11):
        p = p_ref[...]
        jv = (lax.broadcasted_iota(jnp.int32, (1, 8), 1).astype(jnp.float32)
              + 0.5) / 7.0
        x1 = p[:, 0:1]
        y1 = p[:, 1:2]
        x2 = p[:, 2:3]
        y2 = p[:, 3:4]
        gx = x1 + jv * (x2 - x1)
        gy = y1 + jv * (y2 - y1)
        fx = gx / 4.0 - 0.5
        fy = gy / 4.0 - 0.5
        x0f = jnp.floor(fx)
        y0f = jnp.floor(fy)
        wx = jnp.clip(fx - x0f, 0.0, 1.0)
        wy = jnp.clip(fy - y0f, 0.0, 1.0)
        x0 = jnp.clip(x0f, 0, W - 1)
        x1i = jnp.clip(x0f + 1, 0, W - 1)
        y0 = jnp.clip(y0f, 0, W - 1)
        y1i = jnp.clip(y0f + 1, 0, W - 1)
        r = r_ref[...]
        t = t_ref[...]
        y0e = _dot(y0, r)
        y1e = _dot(y1i, r)
        x0e = _dot(x0, t)
        x1e = _dot(x1i, t)
        wye = _dot(wy, r)
        wxe = _dot(wx, t)
        i00[...] = (y0e * 128 + x0e).astype(jnp.int32)
        i01[...] = (y0e * 128 + x1e).astype(jnp.int32)
        i10[...] = (y1e * 128 + x0e).astype(jnp.int32)
        i11[...] = (y1e * 128 + x1e).astype(jnp.int32)
        o00[...] = (1.0 - wye) * (1.0 - wxe)
        o01[...] = (1.0 - wye) * wxe
        o10[...] = wye * (1.0 - wxe)
        o11[...] = wye * wxe

    outs = tuple(jax.ShapeDtypeStruct((POST_NMS, 49), jnp.int32) for _ in range(4)) + \
        tuple(jax.ShapeDtypeStruct((POST_NMS, 49), jnp.float32) for _ in range(4))
    return pl.pallas_call(
        body, out_shape=outs,
    )(props, rmat, tmat)


def _sc_gather(table, idx):
    N = idx.shape[0]
    NW = 32
    chunk = N // NW
    nsub = chunk // 128
    mesh = plsc.VectorSubcoreMesh(core_axis_name="c", subcore_axis_name="s")
    idx2d = idx.reshape(NW, nsub, 128)

    @functools.partial(
        pl.kernel, mesh=mesh,
        out_type=jax.ShapeDtypeStruct((N, C), jnp.float32),
        scratch_types=[
            pltpu.VMEM((nsub, 128), jnp.int32),
            pltpu.VMEM((128, C), jnp.float32),
            pltpu.SemaphoreType.DMA,
        ],
    )
    def kk(table_hbm, idx_hbm, out_hbm, idx_v, buf_v, sem):
        wid = lax.axis_index("s") * 2 + lax.axis_index("c")
        base = wid * chunk
        pltpu.sync_copy(idx_hbm.at[wid], idx_v)
        for j in range(nsub):
            pltpu.async_copy(table_hbm.at[idx_v.at[j]], buf_v, sem).wait()
            pltpu.sync_copy(buf_v, out_hbm.at[pl.ds(base + j * 128, 128), :])

    return kk(table, idx2d)


def _combine_pooled(v00, v01, v10, v11, w00, w01, w10, w11):
    M = v00.shape[0]
    bm = 2048

    def body(a_ref, b_ref, c_ref, d_ref, wa, wb, wc, wd, o_ref):
        o_ref[...] = (a_ref[...] * wa[...] + b_ref[...] * wb[...]
                      + c_ref[...] * wc[...] + d_ref[...] * wd[...])

    vspec = pl.BlockSpec((bm, C), lambda i: (i, 0))
    wspec = pl.BlockSpec((bm, 1), lambda i: (i, 0))
    return pl.pallas_call(
        body,
        grid=(M // bm,),
        in_specs=[vspec, vspec, vspec, vspec, wspec, wspec, wspec, wspec],
        out_specs=vspec,
        out_shape=jax.ShapeDtypeStruct((M, C), jnp.float32),
    )(v00, v01, v10, v11, w00, w01, w10, w11)


def _fc1(pooled, w):
    M = pooled.shape[0]
    ck = 1792
    nk = pooled.shape[1] // ck

    def body(x_ref, w_ref, o_ref):
        d = _dot(x_ref[...], w_ref[...])

        @pl.when(pl.program_id(0) == 0)
        def _():
            o_ref[...] = d

        @pl.when(pl.program_id(0) > 0)
        def _():
            o_ref[...] += d

    return pl.pallas_call(
        body,
        grid=(nk,),
        in_specs=[pl.BlockSpec((M, ck), lambda i: (0, i)),
                  pl.BlockSpec((ck, 1024), lambda i: (i, 0))],
        out_specs=pl.BlockSpec((M, 1024), lambda i: (0, 0)),
        out_shape=jax.ShapeDtypeStruct((M, 1024), jnp.float32),
    )(pooled, w)


def _fc_rest(x1raw, b1, w2, b2, wh, bh):
    def body(x_ref, b1_ref, w2_ref, b2_ref, wh_ref, bh_ref, o_ref):
        x1 = jnp.maximum(x_ref[...] + b1_ref[...], 0.0)
        x2 = jnp.maximum(_dot(x1, w2_ref[...]) + b2_ref[...], 0.0)
        o_ref[...] = _dot(x2, wh_ref[...]) + bh_ref[...]

    return pl.pallas_call(
        body,
        out_shape=jax.ShapeDtypeStruct((x1raw.shape[0], 512), jnp.float32),
    )(x1raw, b1, w2, b2, wh, bh)


def kernel(feat0, feat1, feat2, feat3, feat4, rpn_conv_w, rpn_conv_b,
           rpn_cls_w, rpn_cls_b, rpn_box_w, rpn_box_b, fc1_w, fc1_b,
           fc2_w, fc2_b, cls_w, cls_b, reg_w, reg_b):
    feats = [feat0, feat1, feat2, feat3, feat4]

    w9 = rpn_conv_w.transpose(2, 3, 1, 0).reshape(9 * C, C)
    brow = rpn_conv_b.reshape(1, C)
    wc = rpn_cls_w.reshape(A, C)
    wb = rpn_box_w.reshape(A, 4, C)
    wcb = jnp.zeros((C, 48), jnp.float32)
    bcb = jnp.zeros((1, 48), jnp.float32)
    for a in range(A):
        g = 16 * a
        wcb = wcb.at[:, g].set(wc[a])
        bcb = bcb.at[0, g].set(rpn_cls_b[a])
        for c4 in range(4):
            wcb = wcb.at[:, g + 1 + c4].set(wb[a, c4])
            bcb = bcb.at[0, g + 1 + c4].set(rpn_box_b[a * 4 + c4])

    cand_parts = []
    for li, (f, stride) in enumerate(zip(feats, STRIDES)):
        H, W = _LEVEL_HW[li]
        M = H * W
        xt = f[0].transpose(1, 2, 0)
        xpad = jnp.pad(xt, ((1, 1), (1, 1), (0, 0)))
        cols = [xpad[dy:dy + H, dx:dx + W, :] for dy in range(3) for dx in range(3)]
        x2 = jnp.concatenate(cols, axis=-1).reshape(M, 9 * C)
        anc = _make_anchor_params(H, W, stride)
        bm = min(2048, M)
        rows384 = _rpn_level(x2, w9, brow, wcb, bcb, anc, M, bm)
        rows128 = rows384.reshape(M * A, 128)
        k = _LEVEL_K[li]
        if k < M * A:
            scores2d = rows128[:, 0].reshape(-1, 128)
            Nr = scores2d.shape[0]
            ut = jnp.asarray(np.triu(np.ones((128, 128), np.float32)))
            su = jnp.asarray(np.tril(np.ones((Nr, Nr), np.float32), -1))
            dest = _topk_dest(scores2d, ut, su, k)
            cand = _sc_compact(rows128, dest.reshape(-1), k)[:k, :16]
        else:
            cand = rows128[:, :16]
        cand_parts.append(cand)

    pad = jnp.zeros((_NMS_ROWS * 128 - _NCAND, 16), jnp.float32)
    pad = pad.at[:, 0].set(-jnp.inf)
    cand16 = jnp.concatenate(cand_parts + [pad], axis=0)
    s2d = cand16[:, 0].reshape(_NMS_ROWS, 128)
    x12d = cand16[:, 1].reshape(_NMS_ROWS, 128)
    y12d = cand16[:, 2].reshape(_NMS_ROWS, 128)
    x22d = cand16[:, 3].reshape(_NMS_ROWS, 128)
    y22d = cand16[:, 4].reshape(_NMS_ROWS, 128)
    props, ksc = _nms_pallas(cand16, s2d, x12d, y12d, x22d, y22d)

    rmat = np.zeros((8, 49), np.float32)
    tmat = np.zeros((8, 49), np.float32)
    for i in range(7):
        for j in range(7):
            rmat[i, i * 7 + j] = 1.0
            tmat[j, i * 7 + j] = 1.0
    i00, i01, i10, i11, w00, w01, w10, w11 = _roi_pre(
        props, jnp.asarray(rmat), jnp.asarray(tmat))
    table = feat0[0].transpose(1, 2, 0).reshape(_LEVEL_HW[0][0] * _LEVEL_HW[0][1], C)
    npad = 49152 - POST_NMS * 49

    def flat_idx(ix):
        return jnp.concatenate([ix.reshape(-1), jnp.zeros((npad,), jnp.int32)])

    def flat_w(wv):
        return jnp.concatenate(
            [wv.reshape(-1), jnp.zeros((npad,), jnp.float32)]).reshape(-1, 1)

    v00 = _sc_gather(table, flat_idx(i00))
    v01 = _sc_gather(table, flat_idx(i01))
    v10 = _sc_gather(table, flat_idx(i10))
    v11 = _sc_gather(table, flat_idx(i11))
    pooled_rows = _combine_pooled(v00, v01, v10, v11, flat_w(w00), flat_w(w01),
                                  flat_w(w10), flat_w(w11))
    pooled = pooled_rows[:POST_NMS * 49].reshape(POST_NMS, 49 * C)

    fc1_wp = fc1_w.reshape(C, 49, 1024).transpose(1, 0, 2).reshape(49 * C, 1024)
    x1raw = _fc1(pooled, fc1_wp)
    wh = jnp.pad(jnp.concatenate([cls_w, reg_w], axis=1), ((0, 0), (0, 111)))
    bh = jnp.pad(jnp.concatenate([cls_b, reg_b]), (0, 111)).reshape(1, 512)
    heads = _fc_rest(x1raw, fc1_b.reshape(1, 1024), fc2_w,
                     fc2_b.reshape(1, 1024), wh, bh)
    cls_logits = heads[:, :NUM_CLASSES + 1]
    reg = heads[:, NUM_CLASSES + 1:NUM_CLASSES + 1 + NUM_CLASSES * 4]
    return props, ksc.reshape(POST_NMS), cls_logits, reg
